# baseline (speedup 1.0000x reference)
import functools

import jax
import jax.numpy as jnp
from jax.experimental import pallas as pl
from jax.experimental.pallas import tpu as pltpu

N = 10000
K = 32
C0 = 128
C1 = 64
NKP = 15
EXT = 0.12
B = 4
SEG = N // B

QB = 200
GRID = N // QB

_PREC = jax.lax.Precision.HIGHEST


def _lrelu(x):
    return jnp.where(x >= 0, x, 0.1 * x)


def _kpconv_weights(nx, ny, nz, qx, qy, qz, kp_ref):
    dx = nx - qx
    dy = ny - qy
    dz = nz - qz
    d2 = dx * dx + dy * dy + dz * dz
    ws = []
    for p in range(NKP):
        kx = kp_ref[p, 0]
        ky = kp_ref[p, 1]
        kz = kp_ref[p, 2]
        t = d2 - 2.0 * (dx * kx + dy * ky + dz * kz) + (kx * kx + ky * ky + kz * kz)
        w = jnp.maximum(0.0, 1.0 - jnp.sqrt(t + 1e-12) * (1.0 / EXT))
        ws.append(w)
    return ws


def _weighted_concat(ws, g_ref, c):
    parts = []
    for p in range(NKP):
        acc = jnp.zeros((QB, c), jnp.float32)
        w = ws[p]
        for k in range(K):
            acc = acc + w[:, k:k + 1] * g_ref[:, k * c:(k + 1) * c]
        parts.append(acc)
    return jnp.concatenate(parts, axis=1)


def _block0_body(kp_ref, nx_ref, ny_ref, nz_ref, qx_ref, qy_ref, qz_ref,
                 g_ref, w0_ref, wu1_ref, f0_ref, x_ref):
    ws = _kpconv_weights(nx_ref[...], ny_ref[...], nz_ref[...],
                         qx_ref[...], qy_ref[...], qz_ref[...], kp_ref)
    weighted = _weighted_concat(ws, g_ref, C0)
    out = jnp.dot(weighted, w0_ref[...], preferred_element_type=jnp.float32,
                  precision=_PREC)
    f0 = _lrelu(out)
    f0_ref[...] = f0
    x_ref[...] = _lrelu(jnp.dot(f0, wu1_ref[...],
                                preferred_element_type=jnp.float32,
                                precision=_PREC))


def _block1_body(kp_ref, nx_ref, ny_ref, nz_ref, qx_ref, qy_ref, qz_ref,
                 g_ref, wk1_ref, wu2_ref, f0_ref, out_ref):
    ws = _kpconv_weights(nx_ref[...], ny_ref[...], nz_ref[...],
                         qx_ref[...], qy_ref[...], qz_ref[...], kp_ref)
    weighted = _weighted_concat(ws, g_ref, C1)
    x = _lrelu(jnp.dot(weighted, wk1_ref[...],
                       preferred_element_type=jnp.float32, precision=_PREC))
    x = jnp.dot(x, wu2_ref[...], preferred_element_type=jnp.float32,
                precision=_PREC)
    out_ref[...] = _lrelu(x + f0_ref[...])


def _pool_body(f_ref, out_ref):
    rows = jax.lax.broadcasted_iota(jnp.int32, (8, N), 0)
    cols = jax.lax.broadcasted_iota(jnp.int32, (8, N), 1)
    sel = jnp.where(rows == cols // SEG, 1.0 / SEG, 0.0).astype(jnp.float32)
    out_ref[...] = jnp.dot(sel, f_ref[...], preferred_element_type=jnp.float32,
                           precision=_PREC)


def _row_spec():
    return pl.BlockSpec((QB, K), lambda i: (i, 0))


def _col_spec():
    return pl.BlockSpec((QB, 1), lambda i: (i, 0))


def _full_spec(shape):
    return pl.BlockSpec(shape, lambda i: tuple(0 for _ in shape))


def kernel(points, neighbors, features, stack_lengths, K_points0, W0, Wu1,
           K_points1, Wk1, Wu2):
    del stack_lengths
    ef = neighbors.reshape(-1)
    npts = jnp.take(points, ef, axis=0)
    nx = npts[:, 0].reshape(N, K)
    ny = npts[:, 1].reshape(N, K)
    nz = npts[:, 2].reshape(N, K)
    qx = points[:, 0:1]
    qy = points[:, 1:2]
    qz = points[:, 2:3]

    g0 = jnp.take(features, ef, axis=0).reshape(N, K * C0)
    w0r = W0.reshape(NKP * C0, C0)

    f0, x = pl.pallas_call(
        _block0_body,
        grid=(GRID,),
        in_specs=[
            pl.BlockSpec(memory_space=pltpu.SMEM),
            _row_spec(), _row_spec(), _row_spec(),
            _col_spec(), _col_spec(), _col_spec(),
            pl.BlockSpec((QB, K * C0), lambda i: (i, 0)),
            _full_spec((NKP * C0, C0)),
            _full_spec((C0, C1)),
        ],
        out_specs=[
            pl.BlockSpec((QB, C0), lambda i: (i, 0)),
            pl.BlockSpec((QB, C1), lambda i: (i, 0)),
        ],
        out_shape=[
            jax.ShapeDtypeStruct((N, C0), jnp.float32),
            jax.ShapeDtypeStruct((N, C1), jnp.float32),
        ],
    )(K_points0, nx, ny, nz, qx, qy, qz, g0, w0r, Wu1)

    g1 = jnp.take(x, ef, axis=0).reshape(N, K * C1)
    wk1r = Wk1.reshape(NKP * C1, C1)

    f = pl.pallas_call(
        _block1_body,
        grid=(GRID,),
        in_specs=[
            pl.BlockSpec(memory_space=pltpu.SMEM),
            _row_spec(), _row_spec(), _row_spec(),
            _col_spec(), _col_spec(), _col_spec(),
            pl.BlockSpec((QB, K * C1), lambda i: (i, 0)),
            _full_spec((NKP * C1, C1)),
            _full_spec((C1, C0)),
            pl.BlockSpec((QB, C0), lambda i: (i, 0)),
        ],
        out_specs=pl.BlockSpec((QB, C0), lambda i: (i, 0)),
        out_shape=jax.ShapeDtypeStruct((N, C0), jnp.float32),
    )(K_points1, nx, ny, nz, qx, qy, qz, g1, wk1r, Wu2, f0)

    pooled8 = pl.pallas_call(
        _pool_body,
        grid=(1,),
        in_specs=[_full_spec((N, C0))],
        out_specs=pl.BlockSpec((8, C0), lambda i: (0, 0)),
        out_shape=jax.ShapeDtypeStruct((8, C0), jnp.float32),
    )(f)

    return (f, pooled8[:B])


# speedup vs baseline: 1.7978x; 1.7978x over previous
import jax
import jax.numpy as jnp
from jax.experimental import pallas as pl
from jax.experimental.pallas import tpu as pltpu

N = 10000
K = 32
C0 = 128
C1 = 64
NKP = 15
EXT = 0.12
B = 4
SEG = N // B

QB = 200
GRID = N // QB
G = 8
NG = QB // G

_HI = jax.lax.Precision.HIGHEST


def _lrelu(x):
    return jnp.where(x >= 0, x, 0.1 * x)


def _kpconv_weights(nx, ny, nz, qx, qy, qz, kp_ref):
    dx = nx - qx
    dy = ny - qy
    dz = nz - qz
    d2 = dx * dx + dy * dy + dz * dz
    ws = []
    for p in range(NKP):
        kx = kp_ref[p, 0]
        ky = kp_ref[p, 1]
        kz = kp_ref[p, 2]
        t = d2 - 2.0 * (dx * kx + dy * ky + dz * kz) + (kx * kx + ky * ky + kz * kz)
        w = jnp.maximum(0.0, 1.0 - jnp.sqrt(t + 1e-12) * (1.0 / EXT))
        ws.append(w)
    return ws


def _group_mask():
    rows = jax.lax.broadcasted_iota(jnp.int32, (G, G * K), 0)
    cols = jax.lax.broadcasted_iota(jnp.int32, (G, G * K), 1)
    return jnp.where(rows == cols // K, 1.0, 0.0).astype(jnp.float32)


def _kpconv_mxu(ws, g_ref, wrows_ref, c):
    mask = _group_mask()
    for gi in range(NG):
        q0 = gi * G
        bands = []
        for p in range(NKP):
            wsl = ws[p][q0:q0 + G, :]
            band = jnp.broadcast_to(wsl[:, None, :], (G, G, K)).reshape(G, G * K)
            bands.append(band * mask)
        a = jnp.concatenate(bands, axis=0)
        gg = g_ref[q0 * K:(q0 + G) * K, :]
        o = jnp.dot(a, gg, preferred_element_type=jnp.float32)
        for p in range(NKP):
            wrows_ref[p * QB + q0:p * QB + q0 + G, :] = o[p * G:(p + 1) * G, :]


def _apply_kernel_weights(wrows_ref, w_ref, c, d):
    acc = jnp.zeros((QB, d), jnp.float32)
    for p in range(NKP):
        acc = acc + jnp.dot(wrows_ref[p * QB:(p + 1) * QB, :], w_ref[p],
                            preferred_element_type=jnp.float32)
    return acc


def _block0_body(kp_ref, nx_ref, ny_ref, nz_ref, qx_ref, qy_ref, qz_ref,
                 g_ref, w0_ref, wu1_ref, f0_ref, x_ref, wrows_ref):
    ws = _kpconv_weights(nx_ref[...], ny_ref[...], nz_ref[...],
                         qx_ref[...], qy_ref[...], qz_ref[...], kp_ref)
    _kpconv_mxu(ws, g_ref, wrows_ref, C0)
    out = _apply_kernel_weights(wrows_ref, w0_ref, C0, C0)
    f0 = _lrelu(out)
    f0_ref[...] = f0
    x_ref[...] = _lrelu(jnp.dot(f0, wu1_ref[...],
                                preferred_element_type=jnp.float32))


def _block1_body(kp_ref, nx_ref, ny_ref, nz_ref, qx_ref, qy_ref, qz_ref,
                 g_ref, wk1_ref, wu2_ref, f0_ref, out_ref, wrows_ref):
    ws = _kpconv_weights(nx_ref[...], ny_ref[...], nz_ref[...],
                         qx_ref[...], qy_ref[...], qz_ref[...], kp_ref)
    _kpconv_mxu(ws, g_ref, wrows_ref, C1)
    x = _lrelu(_apply_kernel_weights(wrows_ref, wk1_ref, C1, C1))
    x = jnp.dot(x, wu2_ref[...], preferred_element_type=jnp.float32)
    out_ref[...] = _lrelu(x + f0_ref[...])


def _pool_body(f_ref, out_ref):
    rows = jax.lax.broadcasted_iota(jnp.int32, (8, N), 0)
    cols = jax.lax.broadcasted_iota(jnp.int32, (8, N), 1)
    sel = jnp.where(rows == cols // SEG, 1.0 / SEG, 0.0).astype(jnp.float32)
    out_ref[...] = jnp.dot(sel, f_ref[...], preferred_element_type=jnp.float32,
                           precision=_HI)


def _row_spec():
    return pl.BlockSpec((QB, K), lambda i: (i, 0))


def _col_spec():
    return pl.BlockSpec((QB, 1), lambda i: (i, 0))


def _full_spec(shape):
    return pl.BlockSpec(shape, lambda i: tuple(0 for _ in shape))


def kernel(points, neighbors, features, stack_lengths, K_points0, W0, Wu1,
           K_points1, Wk1, Wu2):
    del stack_lengths
    ef = neighbors.reshape(-1)
    npts = jnp.take(points, ef, axis=0)
    nx = npts[:, 0].reshape(N, K)
    ny = npts[:, 1].reshape(N, K)
    nz = npts[:, 2].reshape(N, K)
    qx = points[:, 0:1]
    qy = points[:, 1:2]
    qz = points[:, 2:3]

    g0 = jnp.take(features, ef, axis=0)

    f0, x = pl.pallas_call(
        _block0_body,
        grid=(GRID,),
        in_specs=[
            pl.BlockSpec(memory_space=pltpu.SMEM),
            _row_spec(), _row_spec(), _row_spec(),
            _col_spec(), _col_spec(), _col_spec(),
            pl.BlockSpec((QB * K, C0), lambda i: (i, 0)),
            _full_spec((NKP, C0, C0)),
            _full_spec((C0, C1)),
        ],
        out_specs=[
            pl.BlockSpec((QB, C0), lambda i: (i, 0)),
            pl.BlockSpec((QB, C1), lambda i: (i, 0)),
        ],
        out_shape=[
            jax.ShapeDtypeStruct((N, C0), jnp.float32),
            jax.ShapeDtypeStruct((N, C1), jnp.float32),
        ],
        scratch_shapes=[pltpu.VMEM((NKP * QB, C0), jnp.float32)],
    )(K_points0, nx, ny, nz, qx, qy, qz, g0, W0, Wu1)

    g1 = jnp.take(x, ef, axis=0)

    f = pl.pallas_call(
        _block1_body,
        grid=(GRID,),
        in_specs=[
            pl.BlockSpec(memory_space=pltpu.SMEM),
            _row_spec(), _row_spec(), _row_spec(),
            _col_spec(), _col_spec(), _col_spec(),
            pl.BlockSpec((QB * K, C1), lambda i: (i, 0)),
            _full_spec((NKP, C1, C1)),
            _full_spec((C1, C0)),
            pl.BlockSpec((QB, C0), lambda i: (i, 0)),
        ],
        out_specs=pl.BlockSpec((QB, C0), lambda i: (i, 0)),
        out_shape=jax.ShapeDtypeStruct((N, C0), jnp.float32),
        scratch_shapes=[pltpu.VMEM((NKP * QB, C1), jnp.float32)],
    )(K_points1, nx, ny, nz, qx, qy, qz, g1, Wk1, Wu2, f0)

    pooled8 = pl.pallas_call(
        _pool_body,
        grid=(1,),
        in_specs=[_full_spec((N, C0))],
        out_specs=pl.BlockSpec((8, C0), lambda i: (0, 0)),
        out_shape=jax.ShapeDtypeStruct((8, C0), jnp.float32),
    )(f)

    return (f, pooled8[:B])


# speedup vs baseline: 2.9080x; 1.6175x over previous
import jax
import jax.numpy as jnp
from jax import lax
from jax.experimental import pallas as pl
from jax.experimental.pallas import tpu as pltpu
from jax.experimental.pallas import tpu_sc as plsc

N = 10000
K = 32
C0 = 128
C1 = 64
NKP = 15
EXT = 0.12
B = 4
SEG = N // B

N2 = 10240
E2 = N2 * K

QB = 320
GRID = N2 // QB
G = 8
NG = QB // G
GK = G * K

NC = 2
NS = 16
NW = NC * NS
EPW = E2 // NW

_HI = jax.lax.Precision.HIGHEST


def _lrelu(x):
    return jnp.where(x >= 0, x, 0.1 * x)


def _sc_pipe(table, out, idx_v, bufs, gsems, ssems, rb, base, nch):
    gh = [None, None]
    sh = [None, None]
    gh[0] = pltpu.async_copy(table.at[idx_v.at[pl.ds(0, rb)]], bufs[0],
                             gsems[0])
    for i in range(nch):
        b = i & 1
        nb = 1 - b
        if i + 1 < nch:
            if sh[nb] is not None:
                sh[nb].wait()
            gh[nb] = pltpu.async_copy(
                table.at[idx_v.at[pl.ds((i + 1) * rb, rb)]], bufs[nb],
                gsems[nb])
        gh[b].wait()
        sh[b] = pltpu.async_copy(bufs[b], out.at[pl.ds(base + i * rb, rb)],
                                 ssems[b])
    for h in sh:
        if h is not None:
            h.wait()


_RB = 256


def _sc_gather0_body(feat_hbm, pts_hbm, idx_hbm, g0_hbm, npt_hbm,
                     idx_v, b0, b1, gs0, gs1, ss0, ss1):
    wid = lax.axis_index("s") * NC + lax.axis_index("c")
    base = wid * EPW
    pltpu.sync_copy(idx_hbm.at[pl.ds(base, EPW)], idx_v)
    _sc_pipe(pts_hbm, npt_hbm, idx_v, [b0, b1], [gs0, gs1], [ss0, ss1],
             _RB, base, EPW // _RB)
    _sc_pipe(feat_hbm, g0_hbm, idx_v, [b0, b1], [gs0, gs1], [ss0, ss1],
             _RB, base, EPW // _RB)


def _sc_gather1_body(x_hbm, idx_hbm, g1_hbm,
                     idx_v, b0, b1, gs0, gs1, ss0, ss1):
    wid = lax.axis_index("s") * NC + lax.axis_index("c")
    base = wid * EPW
    pltpu.sync_copy(idx_hbm.at[pl.ds(base, EPW)], idx_v)
    _sc_pipe(x_hbm, g1_hbm, idx_v, [b0, b1], [gs0, gs1], [ss0, ss1],
             _RB, base, EPW // _RB)


def _gather0(features, pts128, ef):
    mesh = plsc.VectorSubcoreMesh(core_axis_name="c", subcore_axis_name="s")
    return pl.kernel(
        _sc_gather0_body,
        mesh=mesh,
        out_type=[
            jax.ShapeDtypeStruct((E2, C0), jnp.float32),
            jax.ShapeDtypeStruct((E2, C0), jnp.float32),
        ],
        scratch_types=[
            pltpu.VMEM((EPW,), jnp.int32),
            pltpu.VMEM((_RB, C0), jnp.float32),
            pltpu.VMEM((_RB, C0), jnp.float32),
            pltpu.SemaphoreType.DMA,
            pltpu.SemaphoreType.DMA,
            pltpu.SemaphoreType.DMA,
            pltpu.SemaphoreType.DMA,
        ],
    )(features, pts128, ef)


def _gather1(x, ef):
    mesh = plsc.VectorSubcoreMesh(core_axis_name="c", subcore_axis_name="s")
    return pl.kernel(
        _sc_gather1_body,
        mesh=mesh,
        out_type=jax.ShapeDtypeStruct((E2, C0), jnp.float32),
        scratch_types=[
            pltpu.VMEM((EPW,), jnp.int32),
            pltpu.VMEM((_RB, C0), jnp.float32),
            pltpu.VMEM((_RB, C0), jnp.float32),
            pltpu.SemaphoreType.DMA,
            pltpu.SemaphoreType.DMA,
            pltpu.SemaphoreType.DMA,
            pltpu.SemaphoreType.DMA,
        ],
    )(x, ef)


def _kpconv_weights_tiled(nxt, nyt, nzt, qxt, qyt, qzt, kp_ref):
    dx = nxt - qxt
    dy = nyt - qyt
    dz = nzt - qzt
    d2 = dx * dx + dy * dy + dz * dz
    ws = []
    for p in range(NKP):
        kx = kp_ref[p, 0]
        ky = kp_ref[p, 1]
        kz = kp_ref[p, 2]
        t = d2 - 2.0 * (dx * kx + dy * ky + dz * kz) + (kx * kx + ky * ky + kz * kz)
        w = jnp.maximum(0.0, 1.0 - jnp.sqrt(t + 1e-12) * (1.0 / EXT))
        ws.append(w)
    return ws


def _group_mask():
    rows = jax.lax.broadcasted_iota(jnp.int32, (G, GK), 0)
    cols = jax.lax.broadcasted_iota(jnp.int32, (G, GK), 1)
    return jnp.where(rows == cols // K, 1.0, 0.0).astype(jnp.float32)


def _kpconv_mxu(ws, g_ref, wrows_ref):
    mask = _group_mask()
    for gi in range(NG):
        bands = []
        for p in range(NKP):
            row = ws[p][gi:gi + 1, :]
            bands.append(jnp.broadcast_to(row, (G, GK)) * mask)
        a = jnp.concatenate(bands, axis=0)
        gg = g_ref[gi * GK:(gi + 1) * GK, :]
        o = jnp.dot(a, gg, preferred_element_type=jnp.float32)
        for p in range(NKP):
            wrows_ref[p, gi * G:(gi + 1) * G, :] = o[p * G:(p + 1) * G, :]


def _apply_kernel_weights(wrows_ref, w_ref, d):
    acc = jnp.zeros((QB, d), jnp.float32)
    for p in range(NKP):
        acc = acc + jnp.dot(wrows_ref[p], w_ref[p],
                            preferred_element_type=jnp.float32)
    return acc


def _block0_body(kp_ref, nxt_ref, nyt_ref, nzt_ref, qxt_ref, qyt_ref, qzt_ref,
                 g_ref, w0_ref, wu1_ref, f0_ref, x_ref, wrows_ref):
    ws = _kpconv_weights_tiled(nxt_ref[...], nyt_ref[...], nzt_ref[...],
                               qxt_ref[...], qyt_ref[...], qzt_ref[...],
                               kp_ref)
    _kpconv_mxu(ws, g_ref, wrows_ref)
    out = _apply_kernel_weights(wrows_ref, w0_ref, C0)
    f0 = _lrelu(out)
    f0_ref[...] = f0
    x_ref[...] = _lrelu(jnp.dot(f0, wu1_ref[...],
                                preferred_element_type=jnp.float32))


def _block1_body(kp_ref, nxt_ref, nyt_ref, nzt_ref, qxt_ref, qyt_ref, qzt_ref,
                 g_ref, wk1_ref, wu2_ref, f0_ref, out_ref, wrows_ref):
    ws = _kpconv_weights_tiled(nxt_ref[...], nyt_ref[...], nzt_ref[...],
                               qxt_ref[...], qyt_ref[...], qzt_ref[...],
                               kp_ref)
    _kpconv_mxu(ws, g_ref, wrows_ref)
    x = _lrelu(_apply_kernel_weights(wrows_ref, wk1_ref, C1))
    x = jnp.dot(x, wu2_ref[...], preferred_element_type=jnp.float32)
    out_ref[...] = _lrelu(x + f0_ref[...])


def _pool_body(f_ref, out_ref):
    rows = jax.lax.broadcasted_iota(jnp.int32, (8, N2), 0)
    cols = jax.lax.broadcasted_iota(jnp.int32, (8, N2), 1)
    sel = jnp.where(rows == cols // SEG, 1.0 / SEG, 0.0).astype(jnp.float32)
    out_ref[...] = jnp.dot(sel, f_ref[...], preferred_element_type=jnp.float32,
                           precision=_HI)


def _tile_spec():
    return pl.BlockSpec((NG, GK), lambda i: (i, 0))


def _full_spec(shape):
    return pl.BlockSpec(shape, lambda i: tuple(0 for _ in shape))


def _run_block0(K_points0, nxt, nyt, nzt, qxt, qyt, qzt, g0, W0, Wu1):
    return pl.pallas_call(
        _block0_body,
        grid=(GRID,),
        in_specs=[
            pl.BlockSpec(memory_space=pltpu.SMEM),
            _tile_spec(), _tile_spec(), _tile_spec(),
            _tile_spec(), _tile_spec(), _tile_spec(),
            pl.BlockSpec((QB * K, C0), lambda i: (i, 0)),
            _full_spec((NKP, C0, C0)),
            _full_spec((C0, C0)),
        ],
        out_specs=[
            pl.BlockSpec((QB, C0), lambda i: (i, 0)),
            pl.BlockSpec((QB, C0), lambda i: (i, 0)),
        ],
        out_shape=[
            jax.ShapeDtypeStruct((N2, C0), jnp.float32),
            jax.ShapeDtypeStruct((N2, C0), jnp.float32),
        ],
        scratch_shapes=[pltpu.VMEM((NKP, QB, C0), jnp.float32)],
    )(K_points0, nxt, nyt, nzt, qxt, qyt, qzt, g0, W0, Wu1)


def _run_block1(K_points1, nxt, nyt, nzt, qxt, qyt, qzt, g1, Wk1, Wu2, f0):
    return pl.pallas_call(
        _block1_body,
        grid=(GRID,),
        in_specs=[
            pl.BlockSpec(memory_space=pltpu.SMEM),
            _tile_spec(), _tile_spec(), _tile_spec(),
            _tile_spec(), _tile_spec(), _tile_spec(),
            pl.BlockSpec((QB * K, C0), lambda i: (i, 0)),
            _full_spec((NKP, C0, C1)),
            _full_spec((C1, C0)),
            pl.BlockSpec((QB, C0), lambda i: (i, 0)),
        ],
        out_specs=pl.BlockSpec((QB, C0), lambda i: (i, 0)),
        out_shape=jax.ShapeDtypeStruct((N2, C0), jnp.float32),
        scratch_shapes=[pltpu.VMEM((NKP, QB, C0), jnp.float32)],
    )(K_points1, nxt, nyt, nzt, qxt, qyt, qzt, g1, Wk1, Wu2, f0)


def _run_pool(f):
    return pl.pallas_call(
        _pool_body,
        grid=(1,),
        in_specs=[_full_spec((N2, C0))],
        out_specs=pl.BlockSpec((8, C0), lambda i: (0, 0)),
        out_shape=jax.ShapeDtypeStruct((8, C0), jnp.float32),
    )(f)


def kernel(points, neighbors, features, stack_lengths, K_points0, W0, Wu1,
           K_points1, Wk1, Wu2):
    del stack_lengths
    pad = N2 - N
    ef = jnp.pad(neighbors, ((0, pad), (0, 0))).reshape(-1).astype(jnp.int32)
    pts128 = jnp.pad(points, ((0, 0), (0, C0 - 3)))

    g0, npt = _gather0(features, pts128, ef)

    nxt = npt[:, 0].reshape(N2 // G, GK)
    nyt = npt[:, 1].reshape(N2 // G, GK)
    nzt = npt[:, 2].reshape(N2 // G, GK)
    p2 = jnp.pad(points, ((0, pad), (0, 0)))
    qt = jnp.broadcast_to(p2.reshape(N2 // G, G, 1, 3), (N2 // G, G, K, 3))
    qxt = qt[..., 0].reshape(N2 // G, GK)
    qyt = qt[..., 1].reshape(N2 // G, GK)
    qzt = qt[..., 2].reshape(N2 // G, GK)

    wu1p = jnp.pad(Wu1, ((0, 0), (0, C0 - C1)))
    wk1p = jnp.pad(Wk1, ((0, 0), (0, C0 - C1), (0, 0)))
    f0, x = _run_block0(K_points0, nxt, nyt, nzt, qxt, qyt, qzt, g0, W0, wu1p)
    g1 = _gather1(x, ef)
    f = _run_block1(K_points1, nxt, nyt, nzt, qxt, qyt, qzt, g1, wk1p, Wu2, f0)
    pooled8 = _run_pool(f)
    return (f[:N], pooled8[:B])
